# baseline (speedup 1.0000x reference)
import dataclasses
import functools

import jax
import jax.numpy as jnp
from jax import lax
from jax.experimental import pallas as pl
from jax.experimental.pallas import tpu as pltpu
from jax.experimental.pallas import tpu_sc as plsc

_N = 10000
_E = 320000
_D = 128
_NC = 2
_NS = 16
_L = 16
_W = _NC * _NS
_NP = 10240
_CH = 128
_NCH = _E // _CH
_CPW = _NCH // _W
_EPW = _CPW * _CH
_EXTRA = _NCH - _W * _CPW
_DPW = _E // _W
_RPS = _NP // _NS
_ZROWS = 64
_HR = _NP // 128
_HRPS = 8
_HSUB = _HR // _HRPS

_mesh = plsc.VectorSubcoreMesh(core_axis_name="c", subcore_axis_name="s")


def _sc_params():
    cp = pltpu.CompilerParams()
    if "needs_layout_passes" in pltpu.CompilerParams.__dataclass_fields__:
        cp = dataclasses.replace(cp, needs_layout_passes=False)
    return cp


@functools.partial(
    pl.kernel,
    out_type=jax.ShapeDtypeStruct((_NC, _HR, 128), jnp.float32),
    mesh=_mesh,
    scratch_types=[
        pltpu.VMEM((_DPW,), jnp.int32),
        pltpu.VMEM((_HR, 128), jnp.float32),
        pltpu.VMEM((_HRPS, 128), jnp.float32),
        pltpu.VMEM((_HRPS, 128), jnp.float32),
        pltpu.VMEM_SHARED((_NS * _HR, 128), jnp.float32),
    ],
    compiler_params=_sc_params(),
)
def _sc_degree(dst_hbm, out_hbm, idx_v, hist_v, red_v, tmp_v, stage_sh):
    cid = lax.axis_index("c")
    sid = lax.axis_index("s")
    w = cid * _NS + sid

    pltpu.sync_copy(dst_hbm.at[pl.ds(w * _DPW, _DPW)], idx_v)

    @pl.loop(0, _HR)
    def _(i):
        @pl.loop(0, 128 // _L)
        def _(j):
            hist_v[i, pl.ds(j * _L, _L)] = jnp.zeros((_L,), jnp.float32)

    ones = jnp.full((_L,), 1.0, jnp.float32)

    @pl.loop(0, _DPW // _L)
    def _(j):
        idx = idx_v[pl.ds(j * _L, _L)]
        plsc.addupdate_scatter(
            hist_v,
            [lax.shift_right_logical(idx, 7), lax.bitwise_and(idx, 127)],
            ones,
        )

    pltpu.sync_copy(hist_v, stage_sh.at[pl.ds(sid * _HR, _HR)])
    plsc.subcore_barrier()

    @pl.when(sid < _HSUB)
    def _():
        pltpu.sync_copy(stage_sh.at[pl.ds(sid * _HRPS, _HRPS)], red_v)

        @pl.loop(1, _NS)
        def _(k):
            pltpu.sync_copy(
                stage_sh.at[pl.ds(k * _HR + sid * _HRPS, _HRPS)], tmp_v
            )

            @pl.loop(0, _HRPS)
            def _(r):
                @pl.loop(0, 128 // _L)
                def _(j):
                    red_v[r, pl.ds(j * _L, _L)] = (
                        red_v[r, pl.ds(j * _L, _L)]
                        + tmp_v[r, pl.ds(j * _L, _L)]
                    )

        pltpu.sync_copy(red_v, out_hbm.at[cid, pl.ds(sid * _HRPS, _HRPS)])


@functools.partial(
    pl.kernel,
    out_type=jax.ShapeDtypeStruct((_NC, _NP, _D), jnp.float32),
    mesh=_mesh,
    scratch_types=[
        pltpu.VMEM((_CH,), jnp.int32),
        pltpu.VMEM((_CH,), jnp.int32),
        pltpu.VMEM((_CH,), jnp.int32),
        pltpu.VMEM((_CH,), jnp.int32),
        pltpu.VMEM((_CH, _D), jnp.float32),
        pltpu.VMEM((_CH, _D), jnp.float32),
        pltpu.VMEM((_ZROWS, _D), jnp.float32),
        pltpu.VMEM_SHARED((_NP, _D), jnp.float32),
        pltpu.SemaphoreType.DMA,
        pltpu.SemaphoreType.DMA,
        pltpu.SemaphoreType.DMA,
        pltpu.SemaphoreType.DMA,
        pltpu.SemaphoreType.DMA,
        pltpu.SemaphoreType.DMA,
    ],
)
def _sc_aggregate(g_hbm, src_hbm, dst_hbm, out_hbm, is0, is1, id0, id1,
                  rows0, rows1, zbuf_v, acc_sh, sg0, sg1, sd0, sd1, ss0, ss1):
    cid = lax.axis_index("c")
    sid = lax.axis_index("s")
    w = cid * _NS + sid
    base = w * _EPW

    pltpu.sync_copy(src_hbm.at[pl.ds(base, _CH)], is0)
    pltpu.sync_copy(src_hbm.at[pl.ds(base + _CH, _CH)], is1)
    pltpu.async_copy(dst_hbm.at[pl.ds(base, _CH)], id0, sd0)
    pltpu.async_copy(dst_hbm.at[pl.ds(base + _CH, _CH)], id1, sd1)
    pltpu.async_copy(g_hbm.at[is0], rows0, sg0)
    pltpu.async_copy(g_hbm.at[is1], rows1, sg1)

    @pl.loop(0, _ZROWS)
    def _(i):
        @pl.loop(0, _D // _L)
        def _(j):
            zbuf_v[i, pl.ds(j * _L, _L)] = jnp.zeros((_L,), jnp.float32)

    @pl.loop(0, _RPS // _ZROWS)
    def _(i):
        pltpu.sync_copy(
            zbuf_v, acc_sh.at[pl.ds(sid * _RPS + i * _ZROWS, _ZROWS)]
        )

    plsc.subcore_barrier()

    last = _CPW // 2 - 1

    @pl.loop(0, _CPW // 2)
    def _(j):
        pltpu.make_async_copy(g_hbm.at[is0], rows0, sg0).wait()

        @pl.when(j < last)
        def _():
            pltpu.async_copy(
                src_hbm.at[pl.ds(base + (2 * j + 2) * _CH, _CH)], is0, ss0)

        pltpu.make_async_copy(dst_hbm.at[pl.ds(base, _CH)], id0, sd0).wait()
        pltpu.sync_copy(rows0, acc_sh.at[id0], add=True)

        @pl.when(j < last)
        def _():
            pltpu.async_copy(
                dst_hbm.at[pl.ds(base + (2 * j + 2) * _CH, _CH)], id0, sd0)
            pltpu.make_async_copy(
                src_hbm.at[pl.ds(base, _CH)], is0, ss0).wait()
            pltpu.async_copy(g_hbm.at[is0], rows0, sg0)

        pltpu.make_async_copy(g_hbm.at[is1], rows1, sg1).wait()

        @pl.when(j < last)
        def _():
            pltpu.async_copy(
                src_hbm.at[pl.ds(base + (2 * j + 3) * _CH, _CH)], is1, ss1)

        pltpu.make_async_copy(dst_hbm.at[pl.ds(base, _CH)], id1, sd1).wait()
        pltpu.sync_copy(rows1, acc_sh.at[id1], add=True)

        @pl.when(j < last)
        def _():
            pltpu.async_copy(
                dst_hbm.at[pl.ds(base + (2 * j + 3) * _CH, _CH)], id1, sd1)
            pltpu.make_async_copy(
                src_hbm.at[pl.ds(base, _CH)], is1, ss1).wait()
            pltpu.async_copy(g_hbm.at[is1], rows1, sg1)

    @pl.when(w < _EXTRA)
    def _():
        ebase = (_W * _CPW + w) * _CH
        pltpu.sync_copy(src_hbm.at[pl.ds(ebase, _CH)], is0)
        pltpu.sync_copy(dst_hbm.at[pl.ds(ebase, _CH)], id0)
        pltpu.async_copy(g_hbm.at[is0], rows0, sg0)
        pltpu.make_async_copy(g_hbm.at[is0], rows0, sg0).wait()
        pltpu.sync_copy(rows0, acc_sh.at[id0], add=True)

    plsc.subcore_barrier()
    pltpu.sync_copy(
        acc_sh.at[pl.ds(sid * _RPS, _RPS)],
        out_hbm.at[cid, pl.ds(sid * _RPS, _RPS)],
    )


def _tc_project(x, proj_W, conv_W, deg):
    def body(x_ref, pw_ref, cw_ref, dg_ref, g_ref):
        wc = lax.dot_general(
            cw_ref[...], pw_ref[...], (((1,), (0,)), ((), ())),
            preferred_element_type=jnp.float32,
        )
        h = lax.dot_general(
            x_ref[...], wc, (((1,), (1,)), ((), ())),
            preferred_element_type=jnp.float32,
        )
        g_ref[0:_N, :] = h * lax.rsqrt(dg_ref[0:_N, :] + 1.0)
        g_ref[_N:_NP, :] = jnp.zeros((_NP - _N, _D), jnp.float32)

    return pl.pallas_call(
        body,
        out_shape=jax.ShapeDtypeStruct((_NP, _D), jnp.float32),
    )(x, proj_W, conv_W, deg)


def _tc_finish(v_parts, g, deg, b2d):
    def body(vp_ref, g_ref, dg_ref, b_ref, o_ref):
        v = vp_ref[0, 0:_N, :] + vp_ref[1, 0:_N, :] + g_ref[0:_N, :]
        pre = v * lax.rsqrt(dg_ref[0:_N, :] + 1.0) + b_ref[...]
        nrm = jnp.sqrt(jnp.sum(pre * pre, axis=1, keepdims=True))
        o_ref[...] = pre / jnp.maximum(nrm, 1e-12)

    return pl.pallas_call(
        body,
        out_shape=jax.ShapeDtypeStruct((_N, _D), jnp.float32),
    )(v_parts, g, deg, b2d)


def kernel(x, edge_index, proj_W, conv_W, conv_b):
    src = edge_index[0].astype(jnp.int32)
    dst = edge_index[1].astype(jnp.int32)

    hist_parts = _sc_degree(dst)
    deg = (hist_parts[0] + hist_parts[1]).reshape(_NP)[:, None]
    g = _tc_project(x, proj_W, conv_W, deg)
    v_parts = _sc_aggregate(g, src, dst)
    out = _tc_finish(v_parts, g, deg, conv_b.reshape(1, _D))
    return out


# speedup vs baseline: 1.0708x; 1.0708x over previous
import dataclasses
import functools

import jax
import jax.numpy as jnp
from jax import lax
from jax.experimental import pallas as pl
from jax.experimental.pallas import tpu as pltpu
from jax.experimental.pallas import tpu_sc as plsc

_N = 10000
_E = 320000
_D = 128
_NC = 2
_NS = 16
_L = 16
_W = _NC * _NS
_NP = 10240
_CH = 128
_NCH = _E // _CH
_CPW = _NCH // _W
_EPW = _CPW * _CH
_EXTRA = _NCH - _W * _CPW
_DPW = _E // _W
_NPA = 10112
_RPSA = _NPA // _NS
_HR = _NP // 128
_HRPS = 8
_HSUB = _HR // _HRPS

_mesh = plsc.VectorSubcoreMesh(core_axis_name="c", subcore_axis_name="s")


def _sc_params():
    cp = pltpu.CompilerParams()
    if "needs_layout_passes" in pltpu.CompilerParams.__dataclass_fields__:
        cp = dataclasses.replace(cp, needs_layout_passes=False)
    return cp


@functools.partial(
    pl.kernel,
    out_type=jax.ShapeDtypeStruct((_NC, _HR, 128), jnp.float32),
    mesh=_mesh,
    scratch_types=[
        pltpu.VMEM((_DPW,), jnp.int32),
        pltpu.VMEM((_HR, 128), jnp.float32),
        pltpu.VMEM((_HRPS, 128), jnp.float32),
        pltpu.VMEM((_HRPS, 128), jnp.float32),
        pltpu.VMEM_SHARED((_NS * _HR, 128), jnp.float32),
    ],
    compiler_params=_sc_params(),
)
def _sc_degree(dst_hbm, out_hbm, idx_v, hist_v, red_v, tmp_v, stage_sh):
    cid = lax.axis_index("c")
    sid = lax.axis_index("s")
    w = cid * _NS + sid

    pltpu.sync_copy(dst_hbm.at[pl.ds(w * _DPW, _DPW)], idx_v)

    @pl.loop(0, _HR)
    def _(i):
        @pl.loop(0, 128 // _L)
        def _(j):
            hist_v[i, pl.ds(j * _L, _L)] = jnp.zeros((_L,), jnp.float32)

    ones = jnp.full((_L,), 1.0, jnp.float32)

    @pl.loop(0, _DPW // _L)
    def _(j):
        idx = idx_v[pl.ds(j * _L, _L)]
        plsc.addupdate_scatter(
            hist_v,
            [lax.shift_right_logical(idx, 7), lax.bitwise_and(idx, 127)],
            ones,
        )

    pltpu.sync_copy(hist_v, stage_sh.at[pl.ds(sid * _HR, _HR)])
    plsc.subcore_barrier()

    @pl.when(sid < _HSUB)
    def _():
        pltpu.sync_copy(stage_sh.at[pl.ds(sid * _HRPS, _HRPS)], red_v)

        @pl.loop(1, _NS)
        def _(k):
            pltpu.sync_copy(
                stage_sh.at[pl.ds(k * _HR + sid * _HRPS, _HRPS)], tmp_v
            )

            @pl.loop(0, _HRPS)
            def _(r):
                @pl.loop(0, 128 // _L)
                def _(j):
                    red_v[r, pl.ds(j * _L, _L)] = (
                        red_v[r, pl.ds(j * _L, _L)]
                        + tmp_v[r, pl.ds(j * _L, _L)]
                    )

        pltpu.sync_copy(red_v, out_hbm.at[cid, pl.ds(sid * _HRPS, _HRPS)])


@functools.partial(
    pl.kernel,
    out_type=jax.ShapeDtypeStruct((_NC, _NPA, _D), jnp.float32),
    mesh=_mesh,
    scratch_types=[
        [pltpu.VMEM((_CH,), jnp.int32) for _ in range(3)],
        [pltpu.VMEM((_CH,), jnp.int32) for _ in range(3)],
        [pltpu.VMEM((_CH, _D), jnp.float32) for _ in range(3)],
        pltpu.VMEM_SHARED((_NPA, _D), jnp.float32),
        [pltpu.SemaphoreType.DMA for _ in range(3)],
        [pltpu.SemaphoreType.DMA for _ in range(3)],
        [pltpu.SemaphoreType.DMA for _ in range(3)],
    ],
)
def _sc_aggregate(g_hbm, src_hbm, dst_hbm, out_hbm, isv, idv,
                  rows, acc_sh, ss, sd, sg):
    cid = lax.axis_index("c")
    sid = lax.axis_index("s")
    w = cid * _NS + sid
    base = w * _EPW

    def src_chunk(e):
        return src_hbm.at[pl.ds(base + e * _CH, _CH)]

    def dst_chunk(e):
        return dst_hbm.at[pl.ds(base + e * _CH, _CH)]

    @pl.loop(0, _CH)
    def _(i):
        @pl.loop(0, _D // _L)
        def _(j):
            rows[0][i, pl.ds(j * _L, _L)] = jnp.zeros((_L,), jnp.float32)

    @pl.loop(0, 4)
    def _(i):
        pltpu.sync_copy(
            rows[0], acc_sh.at[pl.ds(sid * _RPSA + i * _CH, _CH)]
        )

    pltpu.sync_copy(
        rows[0].at[pl.ds(0, _RPSA - 4 * _CH)],
        acc_sh.at[pl.ds(sid * _RPSA + 4 * _CH, _RPSA - 4 * _CH)],
    )

    for q in range(3):
        pltpu.async_copy(src_chunk(q), isv[q], ss[q])
        pltpu.async_copy(dst_chunk(q), idv[q], sd[q])
    for b in range(3):
        pltpu.make_async_copy(src_chunk(b), isv[b], ss[b]).wait()
        pltpu.async_copy(g_hbm.at[isv[b]], rows[b], sg[b])

    plsc.subcore_barrier()

    @pl.loop(0, _CPW // 3)
    def _(j):
        for b in range(3):
            e = 3 * j + b
            pltpu.make_async_copy(g_hbm.at[isv[b]], rows[b], sg[b]).wait()

            @pl.when(e < _CPW - 3)
            def _():
                pltpu.async_copy(src_chunk(e + 3), isv[b], ss[b])

            pltpu.make_async_copy(dst_chunk(0), idv[b], sd[b]).wait()
            pltpu.sync_copy(rows[b], acc_sh.at[idv[b]], add=True)

            @pl.when(e < _CPW - 3)
            def _():
                pltpu.async_copy(dst_chunk(e + 3), idv[b], sd[b])
                pltpu.make_async_copy(src_chunk(0), isv[b], ss[b]).wait()
                pltpu.async_copy(g_hbm.at[isv[b]], rows[b], sg[b])

    @pl.when(w < _EXTRA)
    def _():
        ebase = (_W * _CPW + w) * _CH
        pltpu.sync_copy(src_hbm.at[pl.ds(ebase, _CH)], isv[0])
        pltpu.sync_copy(dst_hbm.at[pl.ds(ebase, _CH)], idv[0])
        pltpu.async_copy(g_hbm.at[isv[0]], rows[0], sg[0])
        pltpu.make_async_copy(g_hbm.at[isv[0]], rows[0], sg[0]).wait()
        pltpu.sync_copy(rows[0], acc_sh.at[idv[0]], add=True)

    plsc.subcore_barrier()
    pltpu.sync_copy(
        acc_sh.at[pl.ds(sid * _RPSA, _RPSA)],
        out_hbm.at[cid, pl.ds(sid * _RPSA, _RPSA)],
    )


def _tc_project(x, proj_W, conv_W, deg):
    def body(x_ref, pw_ref, cw_ref, dg_ref, g_ref):
        wc = lax.dot_general(
            cw_ref[...], pw_ref[...], (((1,), (0,)), ((), ())),
            preferred_element_type=jnp.float32,
        )
        h = lax.dot_general(
            x_ref[...], wc, (((1,), (1,)), ((), ())),
            preferred_element_type=jnp.float32,
        )
        g_ref[0:_N, :] = h * lax.rsqrt(dg_ref[0:_N, :] + 1.0)
        g_ref[_N:_NP, :] = jnp.zeros((_NP - _N, _D), jnp.float32)

    return pl.pallas_call(
        body,
        out_shape=jax.ShapeDtypeStruct((_NP, _D), jnp.float32),
    )(x, proj_W, conv_W, deg)


def _tc_finish(v_parts, g, deg, b2d):
    def body(vp_ref, g_ref, dg_ref, b_ref, o_ref):
        v = vp_ref[0, 0:_N, :] + vp_ref[1, 0:_N, :] + g_ref[0:_N, :]
        pre = v * lax.rsqrt(dg_ref[0:_N, :] + 1.0) + b_ref[...]
        nrm = jnp.sqrt(jnp.sum(pre * pre, axis=1, keepdims=True))
        o_ref[...] = pre / jnp.maximum(nrm, 1e-12)

    return pl.pallas_call(
        body,
        out_shape=jax.ShapeDtypeStruct((_N, _D), jnp.float32),
    )(v_parts, g, deg, b2d)


def kernel(x, edge_index, proj_W, conv_W, conv_b):
    src = edge_index[0].astype(jnp.int32)
    dst = edge_index[1].astype(jnp.int32)

    hist_parts = _sc_degree(dst)
    deg = (hist_parts[0] + hist_parts[1]).reshape(_NP)[:, None]
    g = _tc_project(x, proj_W, conv_W, deg)
    v_parts = _sc_aggregate(g, src, dst)
    out = _tc_finish(v_parts, g, deg, conv_b.reshape(1, _D))
    return out


# speedup vs baseline: 1.0718x; 1.0010x over previous
import dataclasses
import functools

import jax
import jax.numpy as jnp
from jax import lax
from jax.experimental import pallas as pl
from jax.experimental.pallas import tpu as pltpu
from jax.experimental.pallas import tpu_sc as plsc

_N = 10000
_E = 320000
_D = 128
_NC = 2
_NS = 16
_L = 16
_W = _NC * _NS
_NP = 10240
_CH = 128
_NCH = _E // _CH
_CPW = _NCH // _W
_EPW = _CPW * _CH
_EXTRA = _NCH - _W * _CPW
_DPW = _E // _W
_NPA = 10112
_RPSA = _NPA // _NS
_HR = _NP // 128
_HRPS = 8
_HSUB = _HR // _HRPS

_mesh = plsc.VectorSubcoreMesh(core_axis_name="c", subcore_axis_name="s")


def _sc_params():
    cp = pltpu.CompilerParams()
    if "needs_layout_passes" in pltpu.CompilerParams.__dataclass_fields__:
        cp = dataclasses.replace(cp, needs_layout_passes=False)
    return cp


@functools.partial(
    pl.kernel,
    out_type=jax.ShapeDtypeStruct((_NC, _HR, 128), jnp.float32),
    mesh=_mesh,
    scratch_types=[
        pltpu.VMEM((_DPW,), jnp.int32),
        pltpu.VMEM((_HR, 128), jnp.float32),
        pltpu.VMEM((_HRPS, 128), jnp.float32),
        pltpu.VMEM((_HRPS, 128), jnp.float32),
        pltpu.VMEM_SHARED((_NS * _HR, 128), jnp.float32),
    ],
    compiler_params=_sc_params(),
)
def _sc_degree(dst_hbm, out_hbm, idx_v, hist_v, red_v, tmp_v, stage_sh):
    cid = lax.axis_index("c")
    sid = lax.axis_index("s")
    w = cid * _NS + sid

    pltpu.sync_copy(dst_hbm.at[pl.ds(w * _DPW, _DPW)], idx_v)

    @pl.loop(0, _HR)
    def _(i):
        @pl.loop(0, 128 // _L)
        def _(j):
            hist_v[i, pl.ds(j * _L, _L)] = jnp.zeros((_L,), jnp.float32)

    ones = jnp.full((_L,), 1.0, jnp.float32)

    @pl.loop(0, _DPW // _L)
    def _(j):
        idx = idx_v[pl.ds(j * _L, _L)]
        plsc.addupdate_scatter(
            hist_v,
            [lax.shift_right_logical(idx, 7), lax.bitwise_and(idx, 127)],
            ones,
        )

    pltpu.sync_copy(hist_v, stage_sh.at[pl.ds(sid * _HR, _HR)])
    plsc.subcore_barrier()

    @pl.when(sid < _HSUB)
    def _():
        pltpu.sync_copy(stage_sh.at[pl.ds(sid * _HRPS, _HRPS)], red_v)

        @pl.loop(1, _NS)
        def _(k):
            pltpu.sync_copy(
                stage_sh.at[pl.ds(k * _HR + sid * _HRPS, _HRPS)], tmp_v
            )

            @pl.loop(0, _HRPS)
            def _(r):
                @pl.loop(0, 128 // _L)
                def _(j):
                    red_v[r, pl.ds(j * _L, _L)] = (
                        red_v[r, pl.ds(j * _L, _L)]
                        + tmp_v[r, pl.ds(j * _L, _L)]
                    )

        pltpu.sync_copy(red_v, out_hbm.at[cid, pl.ds(sid * _HRPS, _HRPS)])


@functools.partial(
    pl.kernel,
    out_type=jax.ShapeDtypeStruct((_NC, _NPA, _D), jnp.float32),
    mesh=_mesh,
    scratch_types=[
        [pltpu.VMEM((_CH,), jnp.int32) for _ in range(3)],
        [pltpu.VMEM((_CH,), jnp.int32) for _ in range(3)],
        [pltpu.VMEM((_CH, _D), jnp.float32) for _ in range(3)],
        pltpu.VMEM_SHARED((_NPA, _D), jnp.float32),
        [pltpu.SemaphoreType.DMA for _ in range(3)],
        [pltpu.SemaphoreType.DMA for _ in range(3)],
        [pltpu.SemaphoreType.DMA for _ in range(3)],
    ],
)
def _sc_aggregate(g_hbm, src_hbm, dst_hbm, out_hbm, isv, idv,
                  rows, acc_sh, ss, sd, sg):
    cid = lax.axis_index("c")
    sid = lax.axis_index("s")
    w = cid * _NS + sid
    base = w * _EPW

    def src_chunk(e):
        return src_hbm.at[pl.ds(base + e * _CH, _CH)]

    def dst_chunk(e):
        return dst_hbm.at[pl.ds(base + e * _CH, _CH)]

    @pl.loop(0, _CH)
    def _(i):
        @pl.loop(0, _D // _L)
        def _(j):
            rows[0][i, pl.ds(j * _L, _L)] = jnp.zeros((_L,), jnp.float32)

    @pl.loop(0, 4)
    def _(i):
        pltpu.sync_copy(
            rows[0], acc_sh.at[pl.ds(sid * _RPSA + i * _CH, _CH)]
        )

    pltpu.sync_copy(
        rows[0].at[pl.ds(0, _RPSA - 4 * _CH)],
        acc_sh.at[pl.ds(sid * _RPSA + 4 * _CH, _RPSA - 4 * _CH)],
    )

    for q in range(3):
        pltpu.async_copy(src_chunk(q), isv[q], ss[q])
        pltpu.async_copy(dst_chunk(q), idv[q], sd[q])
    for b in range(3):
        pltpu.make_async_copy(src_chunk(b), isv[b], ss[b]).wait()
        pltpu.async_copy(g_hbm.at[isv[b]], rows[b], sg[b])

    plsc.subcore_barrier()

    @pl.loop(0, _CPW // 3)
    def _(j):
        for b in range(3):
            e = 3 * j + b
            pltpu.make_async_copy(g_hbm.at[isv[b]], rows[b], sg[b]).wait()

            @pl.when(e < _CPW - 3)
            def _():
                pltpu.async_copy(src_chunk(e + 3), isv[b], ss[b])

            pltpu.make_async_copy(dst_chunk(0), idv[b], sd[b]).wait()
            pltpu.sync_copy(rows[b], acc_sh.at[idv[b]], add=True)

            @pl.when(e < _CPW - 3)
            def _():
                pltpu.async_copy(dst_chunk(e + 3), idv[b], sd[b])
                pltpu.make_async_copy(src_chunk(0), isv[b], ss[b]).wait()
                pltpu.async_copy(g_hbm.at[isv[b]], rows[b], sg[b])

    eidx = sid * _NC + cid

    @pl.when(eidx < _EXTRA)
    def _():
        ebase = (_W * _CPW + eidx) * _CH
        pltpu.sync_copy(src_hbm.at[pl.ds(ebase, _CH)], isv[0])
        pltpu.sync_copy(dst_hbm.at[pl.ds(ebase, _CH)], idv[0])
        pltpu.async_copy(g_hbm.at[isv[0]], rows[0], sg[0])
        pltpu.make_async_copy(g_hbm.at[isv[0]], rows[0], sg[0]).wait()
        pltpu.sync_copy(rows[0], acc_sh.at[idv[0]], add=True)

    plsc.subcore_barrier()
    pltpu.sync_copy(
        acc_sh.at[pl.ds(sid * _RPSA, _RPSA)],
        out_hbm.at[cid, pl.ds(sid * _RPSA, _RPSA)],
    )


def _tc_project(x, proj_W, conv_W, deg):
    def body(x_ref, pw_ref, cw_ref, dg_ref, g_ref):
        wc = lax.dot_general(
            cw_ref[...], pw_ref[...], (((1,), (0,)), ((), ())),
            preferred_element_type=jnp.float32,
        )
        h = lax.dot_general(
            x_ref[...], wc, (((1,), (1,)), ((), ())),
            preferred_element_type=jnp.float32,
        )
        g_ref[0:_N, :] = h * lax.rsqrt(dg_ref[0:_N, :] + 1.0)
        g_ref[_N:_NP, :] = jnp.zeros((_NP - _N, _D), jnp.float32)

    return pl.pallas_call(
        body,
        out_shape=jax.ShapeDtypeStruct((_NP, _D), jnp.float32),
    )(x, proj_W, conv_W, deg)


def _tc_finish(v_parts, g, deg, b2d):
    def body(vp_ref, g_ref, dg_ref, b_ref, o_ref):
        v = vp_ref[0, 0:_N, :] + vp_ref[1, 0:_N, :] + g_ref[0:_N, :]
        pre = v * lax.rsqrt(dg_ref[0:_N, :] + 1.0) + b_ref[...]
        nrm = jnp.sqrt(jnp.sum(pre * pre, axis=1, keepdims=True))
        o_ref[...] = pre / jnp.maximum(nrm, 1e-12)

    return pl.pallas_call(
        body,
        out_shape=jax.ShapeDtypeStruct((_N, _D), jnp.float32),
    )(v_parts, g, deg, b2d)


def kernel(x, edge_index, proj_W, conv_W, conv_b):
    src = edge_index[0].astype(jnp.int32)
    dst = edge_index[1].astype(jnp.int32)

    hist_parts = _sc_degree(dst)
    deg = (hist_parts[0] + hist_parts[1]).reshape(_NP)[:, None]
    g = _tc_project(x, proj_W, conv_W, deg)
    v_parts = _sc_aggregate(g, src, dst)
    out = _tc_finish(v_parts, g, deg, conv_b.reshape(1, _D))
    return out


# speedup vs baseline: 1.0931x; 1.0199x over previous
import dataclasses
import functools

import jax
import jax.numpy as jnp
from jax import lax
from jax.experimental import pallas as pl
from jax.experimental.pallas import tpu as pltpu
from jax.experimental.pallas import tpu_sc as plsc

_N = 10000
_E = 320000
_D = 128
_NC = 2
_NS = 16
_L = 16
_W = _NC * _NS
_NP = 10240
_CH = 128
_NCH = _E // _CH
_CPW = _NCH // _W
_EPW = _CPW * _CH
_EXTRA = _NCH - _W * _CPW
_DPW = _E // _W
_NPA = 10112
_RPSA = _NPA // _NS
_HR = _NP // 128
_HRPS = 8
_HSUB = _HR // _HRPS

_mesh = plsc.VectorSubcoreMesh(core_axis_name="c", subcore_axis_name="s")


def _sc_params():
    cp = pltpu.CompilerParams()
    if "needs_layout_passes" in pltpu.CompilerParams.__dataclass_fields__:
        cp = dataclasses.replace(cp, needs_layout_passes=False)
    return cp


@functools.partial(
    pl.kernel,
    out_type=jax.ShapeDtypeStruct((_NC, _HR, 128), jnp.float32),
    mesh=_mesh,
    scratch_types=[
        pltpu.VMEM((_DPW,), jnp.int32),
        pltpu.VMEM((_HR, 128), jnp.float32),
        pltpu.VMEM((_HRPS, 128), jnp.float32),
        pltpu.VMEM((_HRPS, 128), jnp.float32),
        pltpu.VMEM_SHARED((_NS * _HR, 128), jnp.float32),
    ],
    compiler_params=_sc_params(),
)
def _sc_degree(dst_hbm, out_hbm, idx_v, hist_v, red_v, tmp_v, stage_sh):
    cid = lax.axis_index("c")
    sid = lax.axis_index("s")
    w = cid * _NS + sid

    pltpu.sync_copy(dst_hbm.at[pl.ds(w * _DPW, _DPW)], idx_v)

    @pl.loop(0, _HR)
    def _(i):
        @pl.loop(0, 128 // _L)
        def _(j):
            hist_v[i, pl.ds(j * _L, _L)] = jnp.zeros((_L,), jnp.float32)

    ones = jnp.full((_L,), 1.0, jnp.float32)

    @pl.loop(0, _DPW // _L)
    def _(j):
        idx = idx_v[pl.ds(j * _L, _L)]
        plsc.addupdate_scatter(
            hist_v,
            [lax.shift_right_logical(idx, 7), lax.bitwise_and(idx, 127)],
            ones,
        )

    pltpu.sync_copy(hist_v, stage_sh.at[pl.ds(sid * _HR, _HR)])
    plsc.subcore_barrier()

    @pl.when(sid < _HSUB)
    def _():
        pltpu.sync_copy(stage_sh.at[pl.ds(sid * _HRPS, _HRPS)], red_v)

        @pl.loop(1, _NS)
        def _(k):
            pltpu.sync_copy(
                stage_sh.at[pl.ds(k * _HR + sid * _HRPS, _HRPS)], tmp_v
            )

            @pl.loop(0, _HRPS)
            def _(r):
                @pl.loop(0, 128 // _L)
                def _(j):
                    red_v[r, pl.ds(j * _L, _L)] = (
                        red_v[r, pl.ds(j * _L, _L)]
                        + tmp_v[r, pl.ds(j * _L, _L)]
                    )

        pltpu.sync_copy(red_v, out_hbm.at[cid, pl.ds(sid * _HRPS, _HRPS)])


@functools.partial(
    pl.kernel,
    out_type=jax.ShapeDtypeStruct((_NC, _NPA, _D), jnp.float32),
    mesh=_mesh,
    scratch_types=[
        [pltpu.VMEM((_CH,), jnp.int32) for _ in range(3)],
        [pltpu.VMEM((_CH,), jnp.int32) for _ in range(3)],
        [pltpu.VMEM((_CH, _D), jnp.float32) for _ in range(3)],
        pltpu.VMEM_SHARED((_NPA, _D), jnp.float32),
        [pltpu.SemaphoreType.DMA for _ in range(3)],
        [pltpu.SemaphoreType.DMA for _ in range(3)],
        [pltpu.SemaphoreType.DMA for _ in range(3)],
    ],
)
def _sc_aggregate(g_hbm, src_hbm, dst_hbm, out_hbm, isv, idv,
                  rows, acc_sh, ss, sd, sg):
    cid = lax.axis_index("c")
    sid = lax.axis_index("s")
    w = cid * _NS + sid
    base = w * _EPW

    def src_chunk(e):
        return src_hbm.at[pl.ds(base + e * _CH, _CH)]

    def dst_chunk(e):
        return dst_hbm.at[pl.ds(base + e * _CH, _CH)]

    for q in range(3):
        pltpu.async_copy(src_chunk(q), isv[q], ss[q])
        pltpu.async_copy(dst_chunk(q), idv[q], sd[q])

    @pl.loop(0, _CH)
    def _(i):
        @pl.loop(0, _D // _L)
        def _(j):
            rows[2][i, pl.ds(j * _L, _L)] = jnp.zeros((_L,), jnp.float32)

    for b in range(2):
        pltpu.make_async_copy(src_chunk(b), isv[b], ss[b]).wait()
        pltpu.async_copy(g_hbm.at[isv[b]], rows[b], sg[b])

    @pl.loop(0, 4)
    def _(i):
        pltpu.sync_copy(
            rows[2], acc_sh.at[pl.ds(sid * _RPSA + i * _CH, _CH)]
        )

    pltpu.sync_copy(
        rows[2].at[pl.ds(0, _RPSA - 4 * _CH)],
        acc_sh.at[pl.ds(sid * _RPSA + 4 * _CH, _RPSA - 4 * _CH)],
    )

    pltpu.make_async_copy(src_chunk(2), isv[2], ss[2]).wait()
    pltpu.async_copy(g_hbm.at[isv[2]], rows[2], sg[2])

    plsc.subcore_barrier()

    @pl.loop(0, _CPW // 3)
    def _(j):
        for b in range(3):
            e = 3 * j + b
            pltpu.make_async_copy(g_hbm.at[isv[b]], rows[b], sg[b]).wait()

            @pl.when(e < _CPW - 3)
            def _():
                pltpu.async_copy(src_chunk(e + 3), isv[b], ss[b])

            pltpu.make_async_copy(dst_chunk(0), idv[b], sd[b]).wait()
            pltpu.sync_copy(rows[b], acc_sh.at[idv[b]], add=True)

            @pl.when(e < _CPW - 3)
            def _():
                pltpu.async_copy(dst_chunk(e + 3), idv[b], sd[b])
                pltpu.make_async_copy(src_chunk(0), isv[b], ss[b]).wait()
                pltpu.async_copy(g_hbm.at[isv[b]], rows[b], sg[b])

    eidx = sid * _NC + cid

    @pl.when(eidx < _EXTRA)
    def _():
        ebase = (_W * _CPW + eidx) * _CH
        pltpu.sync_copy(src_hbm.at[pl.ds(ebase, _CH)], isv[0])
        pltpu.sync_copy(dst_hbm.at[pl.ds(ebase, _CH)], idv[0])
        pltpu.async_copy(g_hbm.at[isv[0]], rows[0], sg[0])
        pltpu.make_async_copy(g_hbm.at[isv[0]], rows[0], sg[0]).wait()
        pltpu.sync_copy(rows[0], acc_sh.at[idv[0]], add=True)

    plsc.subcore_barrier()
    pltpu.sync_copy(
        acc_sh.at[pl.ds(sid * _RPSA, _RPSA)],
        out_hbm.at[cid, pl.ds(sid * _RPSA, _RPSA)],
    )


def _tc_project(x, proj_W, conv_W, deg):
    def body(x_ref, pw_ref, cw_ref, dg_ref, g_ref):
        wc = lax.dot_general(
            cw_ref[...], pw_ref[...], (((1,), (0,)), ((), ())),
            preferred_element_type=jnp.float32,
        )
        h = lax.dot_general(
            x_ref[...], wc, (((1,), (1,)), ((), ())),
            preferred_element_type=jnp.float32,
        )
        g_ref[0:_N, :] = h * lax.rsqrt(dg_ref[0:_N, :] + 1.0)
        g_ref[_N:_NP, :] = jnp.zeros((_NP - _N, _D), jnp.float32)

    return pl.pallas_call(
        body,
        out_shape=jax.ShapeDtypeStruct((_NP, _D), jnp.float32),
    )(x, proj_W, conv_W, deg)


def _tc_finish(v_parts, g, deg, b2d):
    def body(vp_ref, g_ref, dg_ref, b_ref, o_ref):
        v = vp_ref[0, 0:_N, :] + vp_ref[1, 0:_N, :] + g_ref[0:_N, :]
        pre = v * lax.rsqrt(dg_ref[0:_N, :] + 1.0) + b_ref[...]
        nrm = jnp.sqrt(jnp.sum(pre * pre, axis=1, keepdims=True))
        o_ref[...] = pre / jnp.maximum(nrm, 1e-12)

    return pl.pallas_call(
        body,
        out_shape=jax.ShapeDtypeStruct((_N, _D), jnp.float32),
    )(v_parts, g, deg, b2d)


def kernel(x, edge_index, proj_W, conv_W, conv_b):
    src = edge_index[0].astype(jnp.int32)
    dst = edge_index[1].astype(jnp.int32)

    hist_parts = _sc_degree(dst)
    deg = (hist_parts[0] + hist_parts[1]).reshape(_NP)[:, None]
    g = _tc_project(x, proj_W, conv_W, deg)
    v_parts = _sc_aggregate(g, src, dst)
    out = _tc_finish(v_parts, g, deg, conv_b.reshape(1, _D))
    return out
